# baseline (speedup 1.0000x reference)
import functools

import jax
import jax.numpy as jnp
from jax import lax
from jax.experimental import pallas as pl
from jax.experimental.pallas import tpu as pltpu
from jax.experimental.pallas import tpu_sc as plsc

_SCALE = 8.0
_NC, _NS = 2, 16
_NW = _NC * _NS
_V = 1000000
_D = 64
_BATCH = 16384
_SEQ = 50
_P = 500736
_W = 1024
_NBLK = _P // _W
_C = 128


def _retile_body(lo_ref, hi_ref, out_ref):
    lo = lo_ref[...]
    hi = hi_ref[...]
    out_ref[...] = jnp.concatenate([lo.T, hi.T], axis=1) * _SCALE


def _retile(tt):
    return pl.pallas_call(
        _retile_body,
        grid=(_NBLK,),
        in_specs=[
            pl.BlockSpec((_D, _W), lambda c: (0, c)),
            pl.BlockSpec(
                (_D, _W), lambda c: (0, jnp.minimum(c + _NBLK, _V // _W))
            ),
        ],
        out_specs=pl.BlockSpec((_W, 128), lambda c: (c, 0)),
        out_shape=jax.ShapeDtypeStruct((_P, 128), jnp.float32),
    )(tt, tt)


def _gather_stage(xt, table_c):
    n_chunks = _SEQ * 4
    mesh = plsc.VectorSubcoreMesh(
        core_axis_name="c", subcore_axis_name="s", num_cores=_NC, num_subcores=_NS
    )

    @functools.partial(
        pl.kernel,
        out_type=jax.ShapeDtypeStruct((_SEQ, _D, _BATCH), jnp.float32),
        mesh=mesh,
        scratch_types=[
            pltpu.VMEM((_SEQ, 512), jnp.int32),
            pltpu.VMEM((n_chunks, _C), jnp.int32),
            pltpu.VMEM((_C, _D), jnp.float32),
            pltpu.VMEM((_C, _D), jnp.float32),
            pltpu.VMEM((_C, _D), jnp.float32),
            pltpu.VMEM((_C, _D), jnp.float32),
            pltpu.VMEM((_D, _C), jnp.float32),
            pltpu.VMEM((_D, _C), jnp.float32),
            pltpu.SemaphoreType.DMA,
            pltpu.SemaphoreType.DMA,
            pltpu.SemaphoreType.DMA,
            pltpu.SemaphoreType.DMA,
            pltpu.SemaphoreType.DMA,
            pltpu.SemaphoreType.DMA,
            pltpu.SemaphoreType.DMA,
        ],
        compiler_params=pltpu.CompilerParams(
            use_tc_tiling_on_sc=False, needs_layout_passes=False
        ),
    )
    def k(xt_hbm, tab_hbm, out_hbm, x_v, rid_v, b0, b1, b2, b3, s0, s1,
          xsem, g0, g1, g2, g3, o0, o1):
        buf = (b0, b1, b2, b3)
        stg = (s0, s1)
        gsem = (g0, g1, g2, g3)
        osem = (o0, o1)
        w = lax.axis_index("s") * _NC + lax.axis_index("c")
        bw = w * 512

        pltpu.async_copy(xt_hbm.at[:, pl.ds(bw, 512)], x_v, xsem).wait()

        def rbody(c, _):
            s = lax.shift_right_logical(c, 2)
            base = lax.bitwise_and(c, 3) * _C

            def one(k16):
                v = x_v[s, pl.ds(base + 16 * k16, 16)]
                ge = lax.shift_right_logical(_P - 1 - v, 31)
                rid_v[c, pl.ds(16 * k16, 16)] = 2 * v - ge * (2 * _P - 1)
                return None

            for k16 in range(_C // 16):
                one(k16)
            return 0

        lax.fori_loop(0, n_chunks, rbody, 0)

        def fire(j, slot):
            pltpu.async_copy(tab_hbm.at[rid_v.at[j]], buf[slot], gsem[slot])

        def wait_gather(slot):
            pltpu.make_async_copy(
                tab_hbm.at[pl.ds(0, _C)], buf[slot], gsem[slot]
            ).wait()

        def compute(slot, st):
            for jblk in range(_C // 16):
                rvec = lax.iota(jnp.int32, 16) + (16 * jblk)

                def dgrp(g, _):
                    for kk in range(8):
                        d = g * 8 + kk
                        lvec = jnp.full((16,), 0, jnp.int32) + d
                        vals = plsc.load_gather(buf[slot], [rvec, lvec])
                        stg[st][d, pl.ds(16 * jblk, 16)] = vals
                    return 0

                lax.fori_loop(0, _D // 8, dgrp, 0)

        def put(j, st):
            s = lax.shift_right_logical(j, 2)
            boff = bw + lax.bitwise_and(j, 3) * _C
            pltpu.async_copy(
                stg[st], out_hbm.at[s, :, pl.ds(boff, _C)], osem[st]
            )

        def wait_put(st):
            pltpu.make_async_copy(
                stg[st], out_hbm.at[0, :, pl.ds(0, _C)], osem[st]
            ).wait()

        fire(0, 0)
        fire(1, 1)

        def body(i, _):
            for b in range(4):
                j = 4 * i + b
                st = b & 1
                nslot = (b + 2) % 4

                @pl.when(j >= 2)
                def _():
                    wait_put(st)

                @pl.when(j + 2 < n_chunks)
                def _():
                    fire(j + 2, nslot)

                wait_gather(b)
                compute(b, st)
                put(j, st)
            return 0

        lax.fori_loop(0, n_chunks // 4, body, 0)
        wait_put(0)
        wait_put(1)

    return k(xt, table_c)


def kernel(x, table):
    tt = table.T
    table_c = _retile(tt).reshape(2 * _P, _D)
    xt = x.astype(jnp.int32).T
    out3 = _gather_stage(xt, table_c)
    return out3.transpose(2, 0, 1)


# speedup vs baseline: 2.4358x; 2.4358x over previous
import functools

import jax
import jax.numpy as jnp
from jax import lax
from jax.experimental import layout as jlayout
from jax.experimental import pallas as pl
from jax.experimental.pallas import tpu as pltpu
from jax.experimental.pallas import tpu_sc as plsc

_SCALE = 8.0
_NC, _NS = 2, 16
_NW = _NC * _NS
_V = 1000000
_D = 64
_BATCH = 16384
_SEQ = 50
_B = _BATCH * _SEQ
_P = 500736
_W = 1024
_NBLK = _P // _W
_C = 128


def _retile_body(lo_ref, hi_ref, out_ref):
    lo = lo_ref[...]
    hi = hi_ref[...]
    out_ref[...] = jnp.concatenate([lo.T, hi.T], axis=1) * _SCALE


def _retile(tt):
    return pl.pallas_call(
        _retile_body,
        grid=(_NBLK,),
        in_specs=[
            pl.BlockSpec((_D, _W), lambda c: (0, c)),
            pl.BlockSpec(
                (_D, _W), lambda c: (0, jnp.minimum(c + _NBLK, _V // _W))
            ),
        ],
        out_specs=pl.BlockSpec((_W, 128), lambda c: (c, 0)),
        out_shape=jax.ShapeDtypeStruct((_P, 128), jnp.float32),
    )(tt, tt)


_CBC = 512
_BH = _BATCH // 2


def _out_retile_body(in_ref, out_ref):
    t = in_ref[0]
    out_ref[0] = jnp.concatenate([t[:, :_D].T, t[:, _D:].T], axis=1)


def _out_retile(out3v):
    return pl.pallas_call(
        _out_retile_body,
        grid=(_SEQ,),
        in_specs=[pl.BlockSpec((1, _BH, 2 * _D), lambda s: (s, 0, 0))],
        out_specs=pl.BlockSpec((1, _D, _BATCH), lambda s: (s, 0, 0)),
        out_shape=jax.ShapeDtypeStruct((_SEQ, _D, _BATCH), jnp.float32),
    )(out3v)


def _gather_stage(x2d, table_c):
    n_chunks = _B // (_NW * _C)
    mesh = plsc.VectorSubcoreMesh(
        core_axis_name="c", subcore_axis_name="s", num_cores=_NC, num_subcores=_NS
    )

    @functools.partial(
        pl.kernel,
        out_type=jax.ShapeDtypeStruct((_B // 2, 2 * _D), jnp.float32),
        mesh=mesh,
        scratch_types=[
            pltpu.VMEM((n_chunks, _C), jnp.int32),
            pltpu.VMEM((n_chunks, _C), jnp.int32),
            pltpu.VMEM((_C, _D), jnp.float32),
            pltpu.VMEM((_C, _D), jnp.float32),
            pltpu.VMEM((_C, _D), jnp.float32),
            pltpu.VMEM((_C, _D), jnp.float32),
            pltpu.SemaphoreType.DMA,
            pltpu.SemaphoreType.DMA,
            pltpu.SemaphoreType.DMA,
            pltpu.SemaphoreType.DMA,
            pltpu.SemaphoreType.DMA,
            pltpu.SemaphoreType.DMA,
            pltpu.SemaphoreType.DMA,
            pltpu.SemaphoreType.DMA,
            pltpu.SemaphoreType.DMA,
        ],
        compiler_params=pltpu.CompilerParams(use_tc_tiling_on_sc=False),
    )
    def k(x_hbm, tab_hbm, out_hbm, idx_v, rid_v, b0, b1, b2, b3, sem, g0, g1, g2, g3, o0, o1, o2, o3):
        buf = (b0, b1, b2, b3)
        gsem = (g0, g1, g2, g3)
        osem = (o0, o1, o2, o3)
        w = lax.axis_index("s") * _NC + lax.axis_index("c")
        for s in range(_SEQ):
            pltpu.async_copy(
                x_hbm.at[pl.ds(s * 128 + 4 * w, 4)],
                idx_v.at[pl.ds(4 * s, 4)],
                sem,
            )
        pltpu.make_async_copy(x_hbm.at[pl.ds(0, n_chunks)], idx_v, sem).wait()

        def rbody(i, _):
            def one(k16):
                v = idx_v[i, pl.ds(16 * k16, 16)]
                ge = lax.shift_right_logical(_P - 1 - v, 31)
                rid_v[i, pl.ds(16 * k16, 16)] = 2 * v - ge * (2 * _P - 1)
                return None

            for k16 in range(_C // 16):
                one(k16)
            return 0

        lax.fori_loop(0, n_chunks, rbody, 0)

        def fire(j, slot):
            pltpu.async_copy(tab_hbm.at[rid_v.at[j]], buf[slot], gsem[slot])

        def wait_gather(slot):
            pltpu.make_async_copy(
                tab_hbm.at[pl.ds(0, _C)], buf[slot], gsem[slot]
            ).wait()

        def put(j, slot):
            r0 = (
                lax.shift_right_logical(j, 2) * (_B // (2 * _SEQ))
                + lax.bitwise_and(w, 15) * 512
                + lax.bitwise_and(j, 3) * _C
            )
            half = lax.shift_right_logical(w, 4)

            @pl.when(half == 0)
            def _():
                pltpu.async_copy(
                    buf[slot],
                    out_hbm.at[pl.ds(r0, _C), pl.ds(0, _D)],
                    osem[slot],
                )

            @pl.when(half == 1)
            def _():
                pltpu.async_copy(
                    buf[slot],
                    out_hbm.at[pl.ds(r0, _C), pl.ds(_D, _D)],
                    osem[slot],
                )

        def wait_put(slot):
            pltpu.make_async_copy(
                buf[slot], out_hbm.at[pl.ds(0, _C), pl.ds(0, _D)], osem[slot]
            ).wait()

        fire(0, 0)
        fire(1, 1)

        def body(i, _):
            for b in range(4):
                j = 4 * i + b
                slot = b
                nslot = (b + 2) % 4

                @pl.when(j >= 2)
                def _():
                    wait_put(nslot)

                @pl.when(j + 2 < n_chunks)
                def _():
                    fire(j + 2, nslot)

                wait_gather(slot)
                put(j, slot)
            return 0

        lax.fori_loop(0, n_chunks // 4, body, 0)
        wait_put(2)
        wait_put(3)

    return k(x2d, table_c)


def kernel(x, table):
    tt = table.T
    table_c = _retile(tt).reshape(2 * _P, _D)
    x2d = x.astype(jnp.int32).T.reshape(_B // _C, _C)
    out2 = _gather_stage(x2d, table_c)
    out3 = _out_retile(out2.reshape(_SEQ, _BH, 2 * _D))
    return out3.transpose(2, 0, 1)


# speedup vs baseline: 2.8804x; 1.1825x over previous
import functools

import jax
import jax.numpy as jnp
from jax import lax
from jax.experimental import layout as jlayout
from jax.experimental import pallas as pl
from jax.experimental.pallas import tpu as pltpu
from jax.experimental.pallas import tpu_sc as plsc

_SCALE = 8.0
_NC, _NS = 2, 16
_NW = _NC * _NS
_V = 1000000
_D = 64
_BATCH = 16384
_SEQ = 50
_B = _BATCH * _SEQ
_P = 501760
_W = 2048
_NBLK = _P // _W
_C = 128


def _retile_body(lo_ref, hi_ref, out_ref):
    lo = lo_ref[...]
    hi = hi_ref[...]
    out_ref[...] = jnp.concatenate([lo.T, hi.T], axis=1) * _SCALE


def _retile(tt):
    return pl.pallas_call(
        _retile_body,
        grid=(_NBLK,),
        in_specs=[
            pl.BlockSpec((_D, _W), lambda c: (0, c)),
            pl.BlockSpec(
                (_D, _W), lambda c: (0, jnp.minimum(c + _NBLK, _V // _W))
            ),
        ],
        out_specs=pl.BlockSpec((_W, 128), lambda c: (c, 0)),
        out_shape=jax.ShapeDtypeStruct((_P, 128), jnp.float32),
    )(tt, tt)


_CBC = 512
_BH = _BATCH // 2


def _out_retile_body(in_ref, out_ref):
    t = in_ref[0]
    out_ref[0] = jnp.concatenate([t[:, :_D].T, t[:, _D:].T], axis=1)


def _out_retile(out3v):
    return pl.pallas_call(
        _out_retile_body,
        grid=(_SEQ,),
        in_specs=[pl.BlockSpec((1, _BH, 2 * _D), lambda s: (s, 0, 0))],
        out_specs=pl.BlockSpec((1, _D, _BATCH), lambda s: (s, 0, 0)),
        out_shape=jax.ShapeDtypeStruct((_SEQ, _D, _BATCH), jnp.float32),
    )(out3v)


def _gather_stage(x2d, table_c):
    n_chunks = _B // (_NW * _C)
    mesh = plsc.VectorSubcoreMesh(
        core_axis_name="c", subcore_axis_name="s", num_cores=_NC, num_subcores=_NS
    )

    @functools.partial(
        pl.kernel,
        out_type=jax.ShapeDtypeStruct((_B // 2, 2 * _D), jnp.float32),
        mesh=mesh,
        scratch_types=[
            pltpu.VMEM((n_chunks, _C), jnp.int32),
            pltpu.VMEM((n_chunks, _C), jnp.int32),
            pltpu.VMEM((_C, _D), jnp.float32),
            pltpu.VMEM((_C, _D), jnp.float32),
            pltpu.VMEM((_C, _D), jnp.float32),
            pltpu.VMEM((_C, _D), jnp.float32),
            pltpu.SemaphoreType.DMA,
            pltpu.SemaphoreType.DMA,
            pltpu.SemaphoreType.DMA,
            pltpu.SemaphoreType.DMA,
            pltpu.SemaphoreType.DMA,
            pltpu.SemaphoreType.DMA,
            pltpu.SemaphoreType.DMA,
            pltpu.SemaphoreType.DMA,
            pltpu.SemaphoreType.DMA,
        ],
        compiler_params=pltpu.CompilerParams(use_tc_tiling_on_sc=False),
    )
    def k(x_hbm, tab_hbm, out_hbm, idx_v, rid_v, b0, b1, b2, b3, sem, g0, g1, g2, g3, o0, o1, o2, o3):
        buf = (b0, b1, b2, b3)
        gsem = (g0, g1, g2, g3)
        osem = (o0, o1, o2, o3)
        w = lax.axis_index("s") * _NC + lax.axis_index("c")
        for s in range(_SEQ):
            pltpu.async_copy(
                x_hbm.at[pl.ds(s * 128 + 4 * w, 4)],
                idx_v.at[pl.ds(4 * s, 4)],
                sem,
            )
        pltpu.make_async_copy(x_hbm.at[pl.ds(0, n_chunks)], idx_v, sem).wait()

        def rbody(i, _):
            def one(k16):
                v = idx_v[i, pl.ds(16 * k16, 16)]
                ge = lax.shift_right_logical(_P - 1 - v, 31)
                rid_v[i, pl.ds(16 * k16, 16)] = 2 * v - ge * (2 * _P - 1)
                return None

            for k16 in range(_C // 16):
                one(k16)
            return 0

        lax.fori_loop(0, n_chunks, rbody, 0)

        def fire(j, slot):
            pltpu.async_copy(tab_hbm.at[rid_v.at[j]], buf[slot], gsem[slot])

        def wait_gather(slot):
            pltpu.make_async_copy(
                tab_hbm.at[pl.ds(0, _C)], buf[slot], gsem[slot]
            ).wait()

        def put(j, slot):
            r0 = (
                lax.shift_right_logical(j, 2) * (_B // (2 * _SEQ))
                + lax.bitwise_and(w, 15) * 512
                + lax.bitwise_and(j, 3) * _C
            )
            half = lax.shift_right_logical(w, 4)

            @pl.when(half == 0)
            def _():
                pltpu.async_copy(
                    buf[slot],
                    out_hbm.at[pl.ds(r0, _C), pl.ds(0, _D)],
                    osem[slot],
                )

            @pl.when(half == 1)
            def _():
                pltpu.async_copy(
                    buf[slot],
                    out_hbm.at[pl.ds(r0, _C), pl.ds(_D, _D)],
                    osem[slot],
                )

        def wait_put(slot):
            pltpu.make_async_copy(
                buf[slot], out_hbm.at[pl.ds(0, _C), pl.ds(0, _D)], osem[slot]
            ).wait()

        fire(0, 0)
        fire(1, 1)

        def body(i, _):
            for b in range(4):
                j = 4 * i + b
                slot = b
                nslot = (b + 2) % 4

                @pl.when(j >= 2)
                def _():
                    wait_put(nslot)

                @pl.when(j + 2 < n_chunks)
                def _():
                    fire(j + 2, nslot)

                wait_gather(slot)
                put(j, slot)
            return 0

        lax.fori_loop(0, n_chunks // 4, body, 0)
        wait_put(2)
        wait_put(3)

    return k(x2d, table_c)


def kernel(x, table):
    tt = table.T
    table_c = _retile(tt).reshape(2 * _P, _D)
    x2d = x.astype(jnp.int32).T.reshape(_B // _C, _C)
    out2 = _gather_stage(x2d, table_c)
    out3 = _out_retile(out2.reshape(_SEQ, _BH, 2 * _D))
    return out3.transpose(2, 0, 1)


# speedup vs baseline: 3.1921x; 1.1082x over previous
import functools

import jax
import jax.numpy as jnp
from jax import lax
from jax.experimental import layout as jlayout
from jax.experimental import pallas as pl
from jax.experimental.pallas import tpu as pltpu
from jax.experimental.pallas import tpu_sc as plsc

_SCALE = 8.0
_NC, _NS = 2, 16
_NW = _NC * _NS
_V = 1000000
_D = 64
_BATCH = 16384
_SEQ = 50
_B = _BATCH * _SEQ
_P = 503808
_W = 4096
_NBLK = _P // _W
_C = 128


def _retile_body(lo_ref, hi_ref, out_ref):
    lo = lo_ref[...]
    hi = hi_ref[...]
    out_ref[...] = jnp.concatenate([lo.T, hi.T], axis=1) * _SCALE


def _retile(tt):
    return pl.pallas_call(
        _retile_body,
        grid=(_NBLK,),
        in_specs=[
            pl.BlockSpec((_D, _W), lambda c: (0, c)),
            pl.BlockSpec(
                (_D, _W), lambda c: (0, jnp.minimum(c + _NBLK, _V // _W))
            ),
        ],
        out_specs=pl.BlockSpec((_W, 128), lambda c: (c, 0)),
        out_shape=jax.ShapeDtypeStruct((_P, 128), jnp.float32),
    )(tt, tt)


_CBC = 512
_BH = _BATCH // 2


def _out_retile_body(in_ref, out_ref):
    t = in_ref[0]
    out_ref[0] = jnp.concatenate([t[:, :_D].T, t[:, _D:].T], axis=1)


def _out_retile(out3v):
    return pl.pallas_call(
        _out_retile_body,
        grid=(_SEQ,),
        in_specs=[pl.BlockSpec((1, _BH, 2 * _D), lambda s: (s, 0, 0))],
        out_specs=pl.BlockSpec((1, _D, _BATCH), lambda s: (s, 0, 0)),
        out_shape=jax.ShapeDtypeStruct((_SEQ, _D, _BATCH), jnp.float32),
    )(out3v)


def _gather_stage(x2d, table_c):
    n_chunks = _B // (_NW * _C)
    mesh = plsc.VectorSubcoreMesh(
        core_axis_name="c", subcore_axis_name="s", num_cores=_NC, num_subcores=_NS
    )

    @functools.partial(
        pl.kernel,
        out_type=jax.ShapeDtypeStruct((_B // 2, 2 * _D), jnp.float32),
        mesh=mesh,
        scratch_types=[
            pltpu.VMEM((n_chunks, _C), jnp.int32),
            pltpu.VMEM((n_chunks, _C), jnp.int32),
            pltpu.VMEM((_C, _D), jnp.float32),
            pltpu.VMEM((_C, _D), jnp.float32),
            pltpu.VMEM((_C, _D), jnp.float32),
            pltpu.VMEM((_C, _D), jnp.float32),
            pltpu.SemaphoreType.DMA,
            pltpu.SemaphoreType.DMA,
            pltpu.SemaphoreType.DMA,
            pltpu.SemaphoreType.DMA,
            pltpu.SemaphoreType.DMA,
            pltpu.SemaphoreType.DMA,
            pltpu.SemaphoreType.DMA,
            pltpu.SemaphoreType.DMA,
            pltpu.SemaphoreType.DMA,
        ],
        compiler_params=pltpu.CompilerParams(use_tc_tiling_on_sc=False),
    )
    def k(x_hbm, tab_hbm, out_hbm, idx_v, rid_v, b0, b1, b2, b3, sem, g0, g1, g2, g3, o0, o1, o2, o3):
        buf = (b0, b1, b2, b3)
        gsem = (g0, g1, g2, g3)
        osem = (o0, o1, o2, o3)
        w = lax.axis_index("s") * _NC + lax.axis_index("c")
        for s in range(_SEQ):
            pltpu.async_copy(
                x_hbm.at[pl.ds(s * 128 + 4 * w, 4)],
                idx_v.at[pl.ds(4 * s, 4)],
                sem,
            )
        pltpu.make_async_copy(x_hbm.at[pl.ds(0, n_chunks)], idx_v, sem).wait()

        def rbody(i, _):
            def one(k16):
                v = idx_v[i, pl.ds(16 * k16, 16)]
                ge = lax.shift_right_logical(_P - 1 - v, 31)
                rid_v[i, pl.ds(16 * k16, 16)] = 2 * v - ge * (2 * _P - 1)
                return None

            for k16 in range(_C // 16):
                one(k16)
            return 0

        lax.fori_loop(0, n_chunks, rbody, 0)

        def fire(j, slot):
            pltpu.async_copy(tab_hbm.at[rid_v.at[j]], buf[slot], gsem[slot])

        def wait_gather(slot):
            pltpu.make_async_copy(
                tab_hbm.at[pl.ds(0, _C)], buf[slot], gsem[slot]
            ).wait()

        def put(j, slot):
            r0 = (
                lax.shift_right_logical(j, 2) * (_B // (2 * _SEQ))
                + lax.bitwise_and(w, 15) * 512
                + lax.bitwise_and(j, 3) * _C
            )
            half = lax.shift_right_logical(w, 4)

            @pl.when(half == 0)
            def _():
                pltpu.async_copy(
                    buf[slot],
                    out_hbm.at[pl.ds(r0, _C), pl.ds(0, _D)],
                    osem[slot],
                )

            @pl.when(half == 1)
            def _():
                pltpu.async_copy(
                    buf[slot],
                    out_hbm.at[pl.ds(r0, _C), pl.ds(_D, _D)],
                    osem[slot],
                )

        def wait_put(slot):
            pltpu.make_async_copy(
                buf[slot], out_hbm.at[pl.ds(0, _C), pl.ds(0, _D)], osem[slot]
            ).wait()

        fire(0, 0)
        fire(1, 1)

        def body(i, _):
            for b in range(4):
                j = 4 * i + b
                slot = b
                nslot = (b + 2) % 4

                @pl.when(j >= 2)
                def _():
                    wait_put(nslot)

                @pl.when(j + 2 < n_chunks)
                def _():
                    fire(j + 2, nslot)

                wait_gather(slot)
                put(j, slot)
            return 0

        lax.fori_loop(0, n_chunks // 4, body, 0)
        wait_put(2)
        wait_put(3)

    return k(x2d, table_c)


def kernel(x, table):
    tt = table.T
    table_c = _retile(tt).reshape(2 * _P, _D)
    x2d = x.astype(jnp.int32).T.reshape(_B // _C, _C)
    out2 = _gather_stage(x2d, table_c)
    out3 = _out_retile(out2.reshape(_SEQ, _BH, 2 * _D))
    return out3.transpose(2, 0, 1)


# speedup vs baseline: 3.3867x; 1.0610x over previous
import functools

import jax
import jax.numpy as jnp
from jax import lax
from jax.experimental import layout as jlayout
from jax.experimental import pallas as pl
from jax.experimental.pallas import tpu as pltpu
from jax.experimental.pallas import tpu_sc as plsc

_SCALE = 8.0
_NC, _NS = 2, 16
_NW = _NC * _NS
_V = 1000000
_D = 64
_BATCH = 16384
_SEQ = 50
_B = _BATCH * _SEQ
_P = 507904
_W = 8192
_NBLK = _P // _W
_C = 128


def _retile_body(lo_ref, hi_ref, out_ref):
    lo = lo_ref[...]
    hi = hi_ref[...]
    out_ref[...] = jnp.concatenate([lo.T, hi.T], axis=1) * _SCALE


def _retile(tt):
    return pl.pallas_call(
        _retile_body,
        grid=(_NBLK,),
        in_specs=[
            pl.BlockSpec((_D, _W), lambda c: (0, c)),
            pl.BlockSpec(
                (_D, _W), lambda c: (0, jnp.minimum(c + _NBLK, _V // _W))
            ),
        ],
        out_specs=pl.BlockSpec((_W, 128), lambda c: (c, 0)),
        out_shape=jax.ShapeDtypeStruct((_P, 128), jnp.float32),
    )(tt, tt)


_CBC = 512
_BH = _BATCH // 2


def _out_retile_body(in_ref, out_ref):
    t = in_ref[0]
    out_ref[0] = jnp.concatenate([t[:, :_D].T, t[:, _D:].T], axis=1)


def _out_retile(out3v):
    return pl.pallas_call(
        _out_retile_body,
        grid=(_SEQ,),
        in_specs=[pl.BlockSpec((1, _BH, 2 * _D), lambda s: (s, 0, 0))],
        out_specs=pl.BlockSpec((1, _D, _BATCH), lambda s: (s, 0, 0)),
        out_shape=jax.ShapeDtypeStruct((_SEQ, _D, _BATCH), jnp.float32),
    )(out3v)


def _gather_stage(x2d, table_c):
    n_chunks = _B // (_NW * _C)
    mesh = plsc.VectorSubcoreMesh(
        core_axis_name="c", subcore_axis_name="s", num_cores=_NC, num_subcores=_NS
    )

    @functools.partial(
        pl.kernel,
        out_type=jax.ShapeDtypeStruct((_B // 2, 2 * _D), jnp.float32),
        mesh=mesh,
        scratch_types=[
            pltpu.VMEM((n_chunks, _C), jnp.int32),
            pltpu.VMEM((n_chunks, _C), jnp.int32),
            pltpu.VMEM((_C, _D), jnp.float32),
            pltpu.VMEM((_C, _D), jnp.float32),
            pltpu.VMEM((_C, _D), jnp.float32),
            pltpu.VMEM((_C, _D), jnp.float32),
            pltpu.SemaphoreType.DMA,
            pltpu.SemaphoreType.DMA,
            pltpu.SemaphoreType.DMA,
            pltpu.SemaphoreType.DMA,
            pltpu.SemaphoreType.DMA,
            pltpu.SemaphoreType.DMA,
            pltpu.SemaphoreType.DMA,
            pltpu.SemaphoreType.DMA,
            pltpu.SemaphoreType.DMA,
        ],
        compiler_params=pltpu.CompilerParams(use_tc_tiling_on_sc=False),
    )
    def k(x_hbm, tab_hbm, out_hbm, idx_v, rid_v, b0, b1, b2, b3, sem, g0, g1, g2, g3, o0, o1, o2, o3):
        buf = (b0, b1, b2, b3)
        gsem = (g0, g1, g2, g3)
        osem = (o0, o1, o2, o3)
        w = lax.axis_index("s") * _NC + lax.axis_index("c")
        for s in range(_SEQ):
            pltpu.async_copy(
                x_hbm.at[pl.ds(s * 128 + 4 * w, 4)],
                idx_v.at[pl.ds(4 * s, 4)],
                sem,
            )
        pltpu.make_async_copy(x_hbm.at[pl.ds(0, n_chunks)], idx_v, sem).wait()

        def rbody(i, _):
            def one(k16):
                v = idx_v[i, pl.ds(16 * k16, 16)]
                ge = lax.shift_right_logical(_P - 1 - v, 31)
                rid_v[i, pl.ds(16 * k16, 16)] = 2 * v - ge * (2 * _P - 1)
                return None

            for k16 in range(_C // 16):
                one(k16)
            return 0

        lax.fori_loop(0, n_chunks, rbody, 0)

        def fire(j, slot):
            pltpu.async_copy(tab_hbm.at[rid_v.at[j]], buf[slot], gsem[slot])

        def wait_gather(slot):
            pltpu.make_async_copy(
                tab_hbm.at[pl.ds(0, _C)], buf[slot], gsem[slot]
            ).wait()

        def put(j, slot):
            r0 = (
                lax.shift_right_logical(j, 2) * (_B // (2 * _SEQ))
                + lax.bitwise_and(w, 15) * 512
                + lax.bitwise_and(j, 3) * _C
            )
            half = lax.shift_right_logical(w, 4)

            @pl.when(half == 0)
            def _():
                pltpu.async_copy(
                    buf[slot],
                    out_hbm.at[pl.ds(r0, _C), pl.ds(0, _D)],
                    osem[slot],
                )

            @pl.when(half == 1)
            def _():
                pltpu.async_copy(
                    buf[slot],
                    out_hbm.at[pl.ds(r0, _C), pl.ds(_D, _D)],
                    osem[slot],
                )

        def wait_put(slot):
            pltpu.make_async_copy(
                buf[slot], out_hbm.at[pl.ds(0, _C), pl.ds(0, _D)], osem[slot]
            ).wait()

        fire(0, 0)
        fire(1, 1)

        def body(i, _):
            for b in range(4):
                j = 4 * i + b
                slot = b
                nslot = (b + 2) % 4

                @pl.when(j >= 2)
                def _():
                    wait_put(nslot)

                @pl.when(j + 2 < n_chunks)
                def _():
                    fire(j + 2, nslot)

                wait_gather(slot)
                put(j, slot)
            return 0

        lax.fori_loop(0, n_chunks // 4, body, 0)
        wait_put(2)
        wait_put(3)

    return k(x2d, table_c)


def kernel(x, table):
    tt = table.T
    table_c = _retile(tt).reshape(2 * _P, _D)
    x2d = x.astype(jnp.int32).T.reshape(_B // _C, _C)
    out2 = _gather_stage(x2d, table_c)
    out3 = _out_retile(out2.reshape(_SEQ, _BH, 2 * _D))
    return out3.transpose(2, 0, 1)


# speedup vs baseline: 3.4685x; 1.0241x over previous
import functools

import jax
import jax.numpy as jnp
from jax import lax
from jax.experimental import layout as jlayout
from jax.experimental import pallas as pl
from jax.experimental.pallas import tpu as pltpu
from jax.experimental.pallas import tpu_sc as plsc

_SCALE = 8.0
_NC, _NS = 2, 16
_NW = _NC * _NS
_V = 1000000
_D = 64
_BATCH = 16384
_SEQ = 50
_B = _BATCH * _SEQ
_P = 507904
_W = 16384
_NBLK = _P // _W
_C = 128


def _retile_body(lo_ref, hi_ref, out_ref):
    lo = lo_ref[...]
    hi = hi_ref[...]
    out_ref[...] = jnp.concatenate([lo.T, hi.T], axis=1) * _SCALE


def _retile(tt):
    return pl.pallas_call(
        _retile_body,
        grid=(_NBLK,),
        in_specs=[
            pl.BlockSpec((_D, _W), lambda c: (0, c)),
            pl.BlockSpec(
                (_D, _W), lambda c: (0, jnp.minimum(c + _NBLK, _V // _W))
            ),
        ],
        out_specs=pl.BlockSpec((_W, 128), lambda c: (c, 0)),
        out_shape=jax.ShapeDtypeStruct((_P, 128), jnp.float32),
    )(tt, tt)


_CBC = 512
_BH = _BATCH // 2


def _out_retile_body(in_ref, out_ref):
    t = in_ref[0]
    out_ref[0] = jnp.concatenate([t[:, :_D].T, t[:, _D:].T], axis=1)


def _out_retile(out3v):
    return pl.pallas_call(
        _out_retile_body,
        grid=(_SEQ,),
        in_specs=[pl.BlockSpec((1, _BH, 2 * _D), lambda s: (s, 0, 0))],
        out_specs=pl.BlockSpec((1, _D, _BATCH), lambda s: (s, 0, 0)),
        out_shape=jax.ShapeDtypeStruct((_SEQ, _D, _BATCH), jnp.float32),
    )(out3v)


def _gather_stage(x2d, table_c):
    n_chunks = _B // (_NW * _C)
    mesh = plsc.VectorSubcoreMesh(
        core_axis_name="c", subcore_axis_name="s", num_cores=_NC, num_subcores=_NS
    )

    @functools.partial(
        pl.kernel,
        out_type=jax.ShapeDtypeStruct((_B // 2, 2 * _D), jnp.float32),
        mesh=mesh,
        scratch_types=[
            pltpu.VMEM((n_chunks, _C), jnp.int32),
            pltpu.VMEM((n_chunks, _C), jnp.int32),
            pltpu.VMEM((_C, _D), jnp.float32),
            pltpu.VMEM((_C, _D), jnp.float32),
            pltpu.VMEM((_C, _D), jnp.float32),
            pltpu.VMEM((_C, _D), jnp.float32),
            pltpu.SemaphoreType.DMA,
            pltpu.SemaphoreType.DMA,
            pltpu.SemaphoreType.DMA,
            pltpu.SemaphoreType.DMA,
            pltpu.SemaphoreType.DMA,
            pltpu.SemaphoreType.DMA,
            pltpu.SemaphoreType.DMA,
            pltpu.SemaphoreType.DMA,
            pltpu.SemaphoreType.DMA,
        ],
        compiler_params=pltpu.CompilerParams(use_tc_tiling_on_sc=False),
    )
    def k(x_hbm, tab_hbm, out_hbm, idx_v, rid_v, b0, b1, b2, b3, sem, g0, g1, g2, g3, o0, o1, o2, o3):
        buf = (b0, b1, b2, b3)
        gsem = (g0, g1, g2, g3)
        osem = (o0, o1, o2, o3)
        w = lax.axis_index("s") * _NC + lax.axis_index("c")
        for s in range(_SEQ):
            pltpu.async_copy(
                x_hbm.at[pl.ds(s * 128 + 4 * w, 4)],
                idx_v.at[pl.ds(4 * s, 4)],
                sem,
            )
        pltpu.make_async_copy(x_hbm.at[pl.ds(0, n_chunks)], idx_v, sem).wait()

        def rbody(i, _):
            def one(k16):
                v = idx_v[i, pl.ds(16 * k16, 16)]
                ge = lax.shift_right_logical(_P - 1 - v, 31)
                rid_v[i, pl.ds(16 * k16, 16)] = 2 * v - ge * (2 * _P - 1)
                return None

            for k16 in range(_C // 16):
                one(k16)
            return 0

        lax.fori_loop(0, n_chunks, rbody, 0)

        def fire(j, slot):
            pltpu.async_copy(tab_hbm.at[rid_v.at[j]], buf[slot], gsem[slot])

        def wait_gather(slot):
            pltpu.make_async_copy(
                tab_hbm.at[pl.ds(0, _C)], buf[slot], gsem[slot]
            ).wait()

        def put(j, slot):
            r0 = (
                lax.shift_right_logical(j, 2) * (_B // (2 * _SEQ))
                + lax.bitwise_and(w, 15) * 512
                + lax.bitwise_and(j, 3) * _C
            )
            half = lax.shift_right_logical(w, 4)

            @pl.when(half == 0)
            def _():
                pltpu.async_copy(
                    buf[slot],
                    out_hbm.at[pl.ds(r0, _C), pl.ds(0, _D)],
                    osem[slot],
                )

            @pl.when(half == 1)
            def _():
                pltpu.async_copy(
                    buf[slot],
                    out_hbm.at[pl.ds(r0, _C), pl.ds(_D, _D)],
                    osem[slot],
                )

        def wait_put(slot):
            pltpu.make_async_copy(
                buf[slot], out_hbm.at[pl.ds(0, _C), pl.ds(0, _D)], osem[slot]
            ).wait()

        fire(0, 0)
        fire(1, 1)

        def body(i, _):
            for b in range(4):
                j = 4 * i + b
                slot = b
                nslot = (b + 2) % 4

                @pl.when(j >= 2)
                def _():
                    wait_put(nslot)

                @pl.when(j + 2 < n_chunks)
                def _():
                    fire(j + 2, nslot)

                wait_gather(slot)
                put(j, slot)
            return 0

        lax.fori_loop(0, n_chunks // 4, body, 0)
        wait_put(2)
        wait_put(3)

    return k(x2d, table_c)


def kernel(x, table):
    tt = table.T
    table_c = _retile(tt).reshape(2 * _P, _D)
    x2d = x.astype(jnp.int32).T.reshape(_B // _C, _C)
    out2 = _gather_stage(x2d, table_c)
    out3 = _out_retile(out2.reshape(_SEQ, _BH, 2 * _D))
    return out3.transpose(2, 0, 1)


# speedup vs baseline: 3.4741x; 1.0016x over previous
import functools

import jax
import jax.numpy as jnp
from jax import lax
from jax.experimental import pallas as pl
from jax.experimental.pallas import tpu as pltpu
from jax.experimental.pallas import tpu_sc as plsc

_SCALE = 8.0
_NC, _NS = 2, 16
_NW = _NC * _NS
_V = 1000000
_D = 64
_BATCH = 16384
_SEQ = 50
_B = _BATCH * _SEQ
_P = 507904
_W = 16384
_NBLK = _P // _W
_C = 128


def _retile_body(lo_ref, hi_ref, out_ref):
    lo = lo_ref[...]
    hi = hi_ref[...]
    out_ref[...] = jnp.concatenate([lo.T, hi.T], axis=1) * _SCALE


def _retile(tt):
    return pl.pallas_call(
        _retile_body,
        grid=(_NBLK,),
        in_specs=[
            pl.BlockSpec((_D, _W), lambda c: (0, c)),
            pl.BlockSpec(
                (_D, _W), lambda c: (0, jnp.minimum(c + _NBLK, _V // _W))
            ),
        ],
        out_specs=pl.BlockSpec((_W, 128), lambda c: (c, 0)),
        out_shape=jax.ShapeDtypeStruct((_P, 128), jnp.float32),
    )(tt, tt)


_BH = _BATCH // 2


def _out_retile_body(in_ref, out_ref):
    t = in_ref[0]
    out_ref[0] = jnp.concatenate([t[:, :_D].T, t[:, _D:].T], axis=1)


def _out_retile(out3v):
    return pl.pallas_call(
        _out_retile_body,
        grid=(_SEQ,),
        in_specs=[pl.BlockSpec((1, _BH, 2 * _D), lambda s: (s, 0, 0))],
        out_specs=pl.BlockSpec((1, _D, _BATCH), lambda s: (s, 0, 0)),
        out_shape=jax.ShapeDtypeStruct((_SEQ, _D, _BATCH), jnp.float32),
    )(out3v)


def _gather_stage(x2d, table_c):
    n_chunks = _B // (_NW * _C)
    mesh = plsc.VectorSubcoreMesh(
        core_axis_name="c", subcore_axis_name="s", num_cores=_NC, num_subcores=_NS
    )

    @functools.partial(
        pl.kernel,
        out_type=jax.ShapeDtypeStruct((_B // 2, 2 * _D), jnp.float32),
        mesh=mesh,
        scratch_types=[
            pltpu.VMEM((n_chunks, _C), jnp.int32),
            pltpu.VMEM((n_chunks, _C), jnp.int32),
            pltpu.VMEM((_C, _D), jnp.float32),
            pltpu.VMEM((_C, _D), jnp.float32),
            pltpu.VMEM((_C, _D), jnp.float32),
            pltpu.VMEM((_C, _D), jnp.float32),
            pltpu.SemaphoreType.DMA,
            pltpu.SemaphoreType.DMA,
            pltpu.SemaphoreType.DMA,
            pltpu.SemaphoreType.DMA,
            pltpu.SemaphoreType.DMA,
            pltpu.SemaphoreType.DMA,
            pltpu.SemaphoreType.DMA,
            pltpu.SemaphoreType.DMA,
            pltpu.SemaphoreType.DMA,
        ],
        compiler_params=pltpu.CompilerParams(use_tc_tiling_on_sc=False),
    )
    def k(x_hbm, tab_hbm, out_hbm, idx_v, rid_v, b0, b1, b2, b3, sem, g0, g1, g2, g3, o0, o1, o2, o3):
        buf = (b0, b1, b2, b3)
        gsem = (g0, g1, g2, g3)
        osem = (o0, o1, o2, o3)
        w = lax.axis_index("s") * _NC + lax.axis_index("c")
        for s in range(_SEQ):
            pltpu.async_copy(
                x_hbm.at[pl.ds(s * 128 + 4 * w, 4)],
                idx_v.at[pl.ds(4 * s, 4)],
                sem,
            )
        pltpu.make_async_copy(x_hbm.at[pl.ds(0, n_chunks)], idx_v, sem).wait()

        def rbody(i, _):
            def one(k16):
                v = idx_v[i, pl.ds(16 * k16, 16)]
                ge = lax.shift_right_logical(_P - 1 - v, 31)
                rid_v[i, pl.ds(16 * k16, 16)] = 2 * v - ge * (2 * _P - 1)
                return None

            for k16 in range(_C // 16):
                one(k16)
            return 0

        lax.fori_loop(0, n_chunks, rbody, 0)

        def fire(j, slot):
            pltpu.async_copy(tab_hbm.at[rid_v.at[j]], buf[slot], gsem[slot])

        def wait_gather(slot):
            pltpu.make_async_copy(
                tab_hbm.at[pl.ds(0, _C)], buf[slot], gsem[slot]
            ).wait()

        def put(j, slot):
            r0 = (
                lax.shift_right_logical(j, 2) * (_B // (2 * _SEQ))
                + lax.bitwise_and(w, 15) * 512
                + lax.bitwise_and(j, 3) * _C
            )
            half = lax.shift_right_logical(w, 4)

            @pl.when(half == 0)
            def _():
                pltpu.async_copy(
                    buf[slot],
                    out_hbm.at[pl.ds(r0, _C), pl.ds(0, _D)],
                    osem[slot],
                )

            @pl.when(half == 1)
            def _():
                pltpu.async_copy(
                    buf[slot],
                    out_hbm.at[pl.ds(r0, _C), pl.ds(_D, _D)],
                    osem[slot],
                )

        def wait_put(slot):
            pltpu.make_async_copy(
                buf[slot], out_hbm.at[pl.ds(0, _C), pl.ds(0, _D)], osem[slot]
            ).wait()

        fire(0, 0)
        fire(1, 1)

        def body(i, _):
            for b in range(4):
                j = 4 * i + b
                slot = b
                nslot = (b + 2) % 4

                @pl.when(j >= 2)
                def _():
                    wait_put(nslot)

                @pl.when(j + 2 < n_chunks)
                def _():
                    fire(j + 2, nslot)

                wait_gather(slot)
                put(j, slot)
            return 0

        lax.fori_loop(0, n_chunks // 4, body, 0)
        wait_put(2)
        wait_put(3)

    return k(x2d, table_c)


def kernel(x, table):
    tt = table.T
    table_c = _retile(tt).reshape(2 * _P, _D)
    x2d = x.astype(jnp.int32).T.reshape(_B // _C, _C)
    out2 = _gather_stage(x2d, table_c)
    out3 = _out_retile(out2.reshape(_SEQ, _BH, 2 * _D))
    return out3.transpose(2, 0, 1)
